# baseline (speedup 1.0000x reference)
import functools

import jax
import jax.numpy as jnp
from jax import lax
from jax.experimental import pallas as pl
from jax.experimental.pallas import tpu as pltpu
from jax.experimental.pallas import tpu_sc as plsc

_R, _C = 4096, 256
_NC, _NS, _L = 2, 16, 16
_NW = _NC * _NS
_RW = _R // _NW


@functools.partial(
    pl.kernel,
    out_type=(
        jax.ShapeDtypeStruct((_R, _C), jnp.int32),
        jax.ShapeDtypeStruct((3, _R, _C), jnp.float32),
    ),
    mesh=plsc.VectorSubcoreMesh(core_axis_name="c", subcore_axis_name="s"),
    scratch_types=[
        pltpu.VMEM((_RW, _C), jnp.int32),
        pltpu.VMEM((_RW // 2, _C), jnp.float32),
        pltpu.VMEM((_RW // 2, _C), jnp.float32),
        pltpu.VMEM((_RW // 2, _C), jnp.float32),
        pltpu.VMEM((_RW // 2, _C), jnp.float32),
        pltpu.SemaphoreType.DMA,
        pltpu.SemaphoreType.DMA,
        pltpu.SemaphoreType.DMA,
        pltpu.SemaphoreType.DMA,
        pltpu.SemaphoreType.DMA,
        pltpu.SemaphoreType.DMA,
        pltpu.SemaphoreType.DMA,
        pltpu.SemaphoreType.DMA,
        pltpu.SemaphoreType.DMA,
        pltpu.SemaphoreType.DMA,
    ],
)
def _remap(sp_hbm, coord_hbm, out_hbm, coord_out_hbm,
           sbuf, cb0, cb1, cb2, cb3,
           ssem, osem, ci0, ci1, ci2, ci3, co0, co1, co2, co3):
    cid = lax.axis_index("c")
    sid = lax.axis_index("s")
    wid = sid * _NC + cid
    r0 = wid * _RW
    hr = _RW // 2
    cbuf = [cb0, cb1, cb2, cb3]
    cisem = [ci0, ci1, ci2, ci3]
    cosem = [co0, co1, co2, co3]

    def cchunk(k):
        p, h = k // 2, k % 2
        return p, r0 + h * hr

    s_in = pltpu.make_async_copy(sp_hbm.at[pl.ds(r0, _RW), :], sbuf, ssem)
    s_in.start()
    c_in = []
    for k in range(4):
        p, rows = cchunk(k)
        cp = pltpu.make_async_copy(
            coord_hbm.at[p, pl.ds(rows, hr), :], cbuf[k], cisem[k])
        cp.start()
        c_in.append(cp)

    neg1 = jnp.full((_L,), -1, jnp.int32)

    def step(r, carry):
        for u in range(_C // _L):
            v = sbuf[r, pl.ds(u * _L, _L)]
            w = v - 1
            ok = w.astype(jnp.uint32) < jnp.uint32(8)
            sbuf[r, pl.ds(u * _L, _L)] = jnp.where(ok, w, neg1)
        return carry

    s_in.wait()
    lax.fori_loop(0, _RW, step, 0)
    s_out = pltpu.make_async_copy(sbuf, out_hbm.at[pl.ds(r0, _RW), :], osem)
    s_out.start()

    c_out = []
    for k in range(6):
        slot = k % 4
        p, rows = cchunk(k)
        c_in[k].wait()
        ocp = pltpu.make_async_copy(
            cbuf[slot], coord_out_hbm.at[p, pl.ds(rows, hr), :], cosem[slot])
        ocp.start()
        c_out.append(ocp)
        if k + 4 < 6:
            c_out[k].wait()
            np_, nrows = cchunk(k + 4)
            cp = pltpu.make_async_copy(
                coord_hbm.at[np_, pl.ds(nrows, hr), :], cbuf[slot], cisem[slot])
            cp.start()
            c_in.append(cp)

    for k in range(2, 6):
        c_out[k].wait()
    s_out.wait()


def kernel(species, coordinates):
    coords3 = jnp.transpose(coordinates, (2, 0, 1))
    out, coords_out = _remap(species, coords3)
    return out, jnp.transpose(coords_out, (1, 2, 0))
